# baseline (speedup 1.0000x reference)
import functools
import jax
import jax.numpy as jnp
from jax import lax
from jax.experimental import pallas as pl
from jax.experimental.pallas import tpu as pltpu
from jax.experimental.pallas import tpu_sc as plsc

_N, _U, _I, _E = 1024, 26, 26, 64
_NW = 32
_BPW = _N // _NW


def _sc_body(xu_hbm, xi_hbm, out_hbm, xu_v, xi_v, sem):
    nc = 2
    wid = lax.axis_index("s") * nc + lax.axis_index("c")
    b0 = wid * _BPW
    pltpu.sync_copy(xu_hbm.at[pl.ds(b0, _BPW)], xu_v)
    pltpu.sync_copy(xi_hbm.at[pl.ds(b0, _BPW)], xi_v)
    copies = []
    for j in range(_I):
        copies.append(
            pltpu.async_copy(
                xu_v, out_hbm.at[pl.ds(b0, _BPW), :, pl.ds(j, 1), pl.ds(0, _E)], sem
            )
        )
    for i in range(_U):
        copies.append(
            pltpu.async_copy(
                xi_v, out_hbm.at[pl.ds(b0, _BPW), pl.ds(i, 1), :, pl.ds(_E, _E)], sem
            )
        )
    for c in copies:
        c.wait()


@jax.jit
def kernel(x_user, x_item):
    n, u, e = x_user.shape
    i = x_item.shape[1]
    xu4 = x_user.reshape(n, u, 1, e)
    xi4 = x_item.reshape(n, 1, i, e)
    mesh = plsc.VectorSubcoreMesh(core_axis_name="c", subcore_axis_name="s")
    f = functools.partial(
        pl.kernel,
        mesh=mesh,
        out_type=jax.ShapeDtypeStruct((n, u, i, 2 * e), jnp.float32),
        scratch_types=[
            pltpu.VMEM((_BPW, u, 1, e), jnp.float32),
            pltpu.VMEM((_BPW, 1, i, e), jnp.float32),
            pltpu.SemaphoreType.DMA,
        ],
        compiler_params=pltpu.CompilerParams(use_tc_tiling_on_sc=False),
    )(_sc_body)
    out4 = f(xu4, xi4)
    return out4.reshape(n, u * i, 2 * e)


# speedup vs baseline: 1.6333x; 1.6333x over previous
import functools
import jax
import jax.numpy as jnp
from jax import lax
from jax.experimental import pallas as pl
from jax.experimental.pallas import tpu as pltpu
from jax.experimental.pallas import tpu_sc as plsc

_N, _U, _I, _E = 1024, 26, 26, 64
_NW = 32
_BPW = _N // _NW
_ROWS = _U * _I


def _sc_body(xu_hbm, xi_hbm, out_hbm, xu_v, xi_v, buf, sem):
    nc = 2
    wid = lax.axis_index("s") * nc + lax.axis_index("c")
    b0 = wid * _BPW

    pltpu.sync_copy(xu_hbm.at[b0], xu_v)
    pltpu.sync_copy(xi_hbm.at[b0], xi_v)

    def batch_body(t, _):
        b = b0 + t
        for i in range(_U):
            u = [xu_v[i, pl.ds(16 * k, 16)] for k in range(4)]

            def jbody(j, _, i=i, u=u):
                row = 26 * i + j
                for k in range(4):
                    buf[row, pl.ds(16 * k, 16)] = u[k]
                for k in range(4):
                    buf[row, pl.ds(64 + 16 * k, 16)] = xi_v[j, pl.ds(16 * k, 16)]
                return None

            lax.fori_loop(0, _I, jbody, None, unroll=2)

        copy = pltpu.async_copy(buf, out_hbm.at[b], sem)
        @pl.when(t < _BPW - 1)
        def _stage():
            pltpu.sync_copy(xu_hbm.at[b + 1], xu_v)
            pltpu.sync_copy(xi_hbm.at[b + 1], xi_v)

        copy.wait()
        return None

    lax.fori_loop(0, _BPW, batch_body, None)


@jax.jit
def kernel(x_user, x_item):
    n, u, e = x_user.shape
    i = x_item.shape[1]
    mesh = plsc.VectorSubcoreMesh(core_axis_name="c", subcore_axis_name="s")
    f = functools.partial(
        pl.kernel,
        mesh=mesh,
        out_type=jax.ShapeDtypeStruct((n, u * i, 2 * e), jnp.float32),
        scratch_types=[
            pltpu.VMEM((u, e), jnp.float32),
            pltpu.VMEM((i, e), jnp.float32),
            pltpu.VMEM((u * i, 2 * e), jnp.float32),
            pltpu.SemaphoreType.DMA,
        ],
    )(_sc_body)
    return f(x_user, x_item)
